# baseline (device time: 313908 ns/iter reference)
import jax

jax.config.update("jax_compilation_cache_dir", "/tmp/scband_jax_cache")
jax.config.update("jax_persistent_cache_min_compile_time_secs", 0.0)
jax.config.update("jax_persistent_cache_min_entry_size_bytes", 0)

import jax.numpy as jnp
from jax import lax
from jax.experimental import pallas as pl
from jax.experimental.pallas import tpu as pltpu

N_DEV = 4
E_LOCAL = 8
T = 2048
D = 1024
E_HALF = 4
KH = E_HALF * D
TT = 128
N_TILES = T // TT


def kernel(x, router_W, route_idx, expert_W):
    scores = jnp.dot(x, router_W, precision=lax.Precision.HIGHEST)
    m = jnp.max(scores, axis=1, keepdims=True)
    e = jnp.exp(scores - m)
    probs = e / jnp.sum(e, axis=1, keepdims=True)
    eids = jnp.arange(32, dtype=jnp.int32)[None, :]
    p0 = jnp.sum(jnp.where(route_idx[:, 0:1] == eids, probs, 0.0),
                 axis=1, keepdims=True)
    p1 = jnp.sum(jnp.where(route_idx[:, 1:2] == eids, probs, 0.0),
                 axis=1, keepdims=True)
    ps = p0 + p1
    packed = jnp.concatenate(
        [p0 / ps, p1 / ps, route_idx.astype(jnp.float32)], axis=1
    ).astype(jnp.bfloat16)

    x_bf = x.astype(jnp.bfloat16)
    ew_bf = expert_W.astype(jnp.bfloat16).reshape(E_LOCAL * D, D)

    def body(x_ref, pk_ref, ew_ref, out_ref,
             x_all, pk_all, w_half, xw, acc_bf, pacc, precv,
             ag_send, ag_recv, p_send, p_recv, load_sem):
        my = lax.axis_index("i")
        left = lax.rem(my + N_DEV - 1, N_DEV)
        right = lax.rem(my + 1, N_DEV)

        def w_load(h):
            return pltpu.make_async_copy(
                ew_ref.at[pl.ds(h * KH, KH), :], w_half, load_sem)

        w_load(0).start()

        barrier = pltpu.get_barrier_semaphore()
        for nbr in (left, right):
            pl.semaphore_signal(barrier, inc=1, device_id=(nbr,),
                                device_id_type=pl.DeviceIdType.MESH)
        pl.semaphore_wait(barrier, 2)

        def start_hop(h):
            rs = []
            for src, dst, k in (
                (x_ref if h == 0 else x_all.at[h - 1], x_all.at[h], 0),
                (pk_ref if h == 0 else pk_all.at[h - 1], pk_all.at[h], 1),
            ):
                r = pltpu.make_async_remote_copy(
                    src_ref=src, dst_ref=dst,
                    send_sem=ag_send.at[h, k], recv_sem=ag_recv.at[h, k],
                    device_id=(right,), device_id_type=pl.DeviceIdType.MESH,
                )
                r.start()
                rs.append(r)
            return rs

        def compute_stage(s, halves, wait_first=False):
            xs_ref = x_ref if s < 0 else x_all.at[s]
            pk_s = pk_ref if s < 0 else pk_all.at[s]
            for hi, h in enumerate(halves):
                if hi > 0:
                    w_load(h).start()
                    w_load(h).wait()
                elif wait_first:
                    w_load(h).wait()

                def build(t, b, h=h):
                    rows = pl.ds(t * TT, TT)
                    xs_t = xs_ref[rows, :]
                    pk_t = pk_s[rows, :]
                    for jj in range(E_HALF):
                        e_f = (my * E_LOCAL + h * E_HALF + jj).astype(
                            jnp.bfloat16)
                        w = (jnp.where(pk_t[:, 2:3] == e_f, pk_t[:, 0:1], 0)
                             + jnp.where(pk_t[:, 3:4] == e_f, pk_t[:, 1:2],
                                         0))
                        xw[b, :, jj * D:(jj + 1) * D] = xs_t * w

                build(0, 0)

                def tile_step(t, _, hi=hi):
                    b = lax.rem(t, 2)

                    @pl.when(t + 1 < N_TILES)
                    def _prefetch():
                        build(t + 1, 1 - b)

                    y = jnp.dot(xw[b], w_half[...],
                                preferred_element_type=jnp.float32)
                    rows = pl.ds(t * TT, TT)
                    if hi == 0:
                        acc_bf[rows, :] = y.astype(jnp.bfloat16)
                    else:
                        tot = (acc_bf[rows, :].astype(jnp.float32)
                               + y).astype(jnp.bfloat16)
                        if s < 0:
                            out_ref[rows, :] = tot
                        else:
                            pacc[s, rows, :] = tot
                    return _

                lax.fori_loop(0, N_TILES, tile_step, 0)

        def start_partial(s):
            owner = lax.rem(my + N_DEV - 1 - s, N_DEV)
            r = pltpu.make_async_remote_copy(
                src_ref=pacc.at[s], dst_ref=precv.at[s],
                send_sem=p_send.at[s], recv_sem=p_recv.at[s],
                device_id=(owner,), device_id_type=pl.DeviceIdType.MESH,
            )
            r.start()
            return r

        hop = start_hop(0)
        compute_stage(-1, (0, 1), wait_first=True)
        partials = []
        for s in range(N_DEV - 1):
            for r in hop:
                r.wait()
            if s < N_DEV - 2:
                hop = start_hop(s + 1)
            compute_stage(s, (1, 0) if s % 2 == 0 else (0, 1))
            partials.append(start_partial(s))
        for r in partials:
            r.wait()

        out_ref[...] = (
            out_ref[...].astype(jnp.float32)
            + precv[0].astype(jnp.float32)
            + precv[1].astype(jnp.float32)
            + precv[2].astype(jnp.float32)
        ).astype(jnp.bfloat16)

    out_bf = pl.pallas_call(
        body,
        out_shape=jax.ShapeDtypeStruct((T, D), jnp.bfloat16),
        in_specs=[
            pl.BlockSpec(memory_space=pltpu.VMEM),
            pl.BlockSpec(memory_space=pltpu.VMEM),
            pl.BlockSpec(memory_space=pltpu.MemorySpace.HBM),
        ],
        out_specs=pl.BlockSpec(memory_space=pltpu.VMEM),
        scratch_shapes=[
            pltpu.VMEM((N_DEV - 1, T, D), jnp.bfloat16),
            pltpu.VMEM((N_DEV - 1, T, 4), jnp.bfloat16),
            pltpu.VMEM((KH, D), jnp.bfloat16),
            pltpu.VMEM((2, TT, KH), jnp.bfloat16),
            pltpu.VMEM((T, D), jnp.bfloat16),
            pltpu.VMEM((N_DEV - 1, T, D), jnp.bfloat16),
            pltpu.VMEM((N_DEV - 1, T, D), jnp.bfloat16),
            pltpu.SemaphoreType.DMA((N_DEV - 1, 2)),
            pltpu.SemaphoreType.DMA((N_DEV - 1, 2)),
            pltpu.SemaphoreType.DMA((N_DEV - 1,)),
            pltpu.SemaphoreType.DMA((N_DEV - 1,)),
            pltpu.SemaphoreType.DMA,
        ],
        compiler_params=pltpu.CompilerParams(
            collective_id=0,
            vmem_limit_bytes=128 * 1024 * 1024,
        ),
    )(x_bf, packed, ew_bf)
    return out_bf.astype(jnp.float32)


# device time: 286990 ns/iter; 1.0938x vs baseline; 1.0938x over previous
import jax

jax.config.update("jax_compilation_cache_dir", "/tmp/scband_jax_cache")
jax.config.update("jax_persistent_cache_min_compile_time_secs", 0.0)
jax.config.update("jax_persistent_cache_min_entry_size_bytes", 0)

import jax.numpy as jnp
from jax import lax
from jax.experimental import pallas as pl
from jax.experimental.pallas import tpu as pltpu

N_DEV = 4
E_LOCAL = 8
T = 2048
D = 1024
E_HALF = 4
KH = E_HALF * D
TT = 128
N_TILES = T // TT


def kernel(x, router_W, route_idx, expert_W):
    scores = jnp.dot(x, router_W, precision=lax.Precision.HIGHEST)
    m = jnp.max(scores, axis=1, keepdims=True)
    e = jnp.exp(scores - m)
    probs = e / jnp.sum(e, axis=1, keepdims=True)
    eids = jnp.arange(32, dtype=jnp.int32)[None, :]
    p0 = jnp.sum(jnp.where(route_idx[:, 0:1] == eids, probs, 0.0),
                 axis=1, keepdims=True)
    p1 = jnp.sum(jnp.where(route_idx[:, 1:2] == eids, probs, 0.0),
                 axis=1, keepdims=True)
    ps = p0 + p1
    packed = jnp.concatenate(
        [p0 / ps, p1 / ps, route_idx.astype(jnp.float32)], axis=1
    ).astype(jnp.bfloat16)

    x_bf = x.astype(jnp.bfloat16)
    ew_bf = expert_W.astype(jnp.bfloat16).reshape(E_LOCAL * D, D)

    def body(x_ref, pk_ref, ew_ref, out_ref,
             x_all, pk_all, w_all, xw, pacc, pscale, precv,
             pscale_recv, ag_send, ag_recv, p_send, p_recv, load_sems):
        my = lax.axis_index("i")
        left = lax.rem(my + N_DEV - 1, N_DEV)
        right = lax.rem(my + 1, N_DEV)

        def w_load(h):
            return pltpu.make_async_copy(
                ew_ref.at[pl.ds(h * KH, KH), :],
                w_all.at[pl.ds(h * KH, KH), :],
                load_sems.at[h])

        w_load(0).start()
        w_load(1).start()

        barrier = pltpu.get_barrier_semaphore()
        for nbr in (left, right):
            pl.semaphore_signal(barrier, inc=1, device_id=(nbr,),
                                device_id_type=pl.DeviceIdType.MESH)
        pl.semaphore_wait(barrier, 2)

        def start_gather(row, srcs, dst_slot, dev):
            rs = []
            for k, (src, dst_arr) in enumerate(
                    zip(srcs, (x_all, pk_all))):
                r = pltpu.make_async_remote_copy(
                    src_ref=src, dst_ref=dst_arr.at[dst_slot],
                    send_sem=ag_send.at[row, k], recv_sem=ag_recv.at[row, k],
                    device_id=(dev,), device_id_type=pl.DeviceIdType.MESH,
                )
                r.start()
                rs.append(r)
            return rs

        loads_waited = [False]

        def compute_stage(s):
            xs_ref = x_ref if s < 0 else x_all.at[s]
            pk_s = pk_ref if s < 0 else pk_all.at[s]
            if not loads_waited[0]:
                w_load(0).wait()
                w_load(1).wait()
                loads_waited[0] = True

            def build(t, b):
                rows = pl.ds(t * TT, TT)
                xs_t = xs_ref[rows, :]
                pk_t = pk_s[rows, :]
                for jj in range(E_LOCAL):
                    e_f = (my * E_LOCAL + jj).astype(jnp.bfloat16)
                    w = (jnp.where(pk_t[:, 2:3] == e_f, pk_t[:, 0:1], 0)
                         + jnp.where(pk_t[:, 3:4] == e_f, pk_t[:, 1:2],
                                     0))
                    xw[b, :, jj * D:(jj + 1) * D] = xs_t * w

            build(0, 0)

            def tile_step(t, _):
                b = lax.rem(t, 2)

                @pl.when(t + 1 < N_TILES)
                def _prefetch():
                    build(t + 1, 1 - b)

                y = jnp.dot(xw[b], w_all[...],
                            preferred_element_type=jnp.float32)
                rows = pl.ds(t * TT, TT)
                if s < 0:
                    out_ref[rows, :] = y.astype(jnp.bfloat16)
                else:
                    amax = jnp.max(jnp.abs(y), axis=1, keepdims=True)
                    sc = (jnp.maximum(amax, 1e-20) / 127.0).astype(
                        jnp.bfloat16)
                    q = jnp.clip(
                        jnp.round(y / sc.astype(jnp.float32)),
                        -127.0, 127.0)
                    pacc[s, rows, :] = q.astype(jnp.int8)
                    pscale[s, rows, :] = sc
                return _

            lax.fori_loop(0, N_TILES, tile_step, 0)

        def start_partial(src_slot, dst_slot, owner):
            rs = []
            for k, (src_arr, dst_arr) in enumerate(
                    ((pacc, precv), (pscale, pscale_recv))):
                r = pltpu.make_async_remote_copy(
                    src_ref=src_arr.at[src_slot],
                    dst_ref=dst_arr.at[dst_slot],
                    send_sem=p_send.at[dst_slot, k],
                    recv_sem=p_recv.at[dst_slot, k],
                    device_id=(owner,), device_id_type=pl.DeviceIdType.MESH,
                )
                r.start()
                rs.append(r)
            return rs

        diag = lax.rem(my + 2, N_DEV)
        h1r = start_gather(0, (x_ref, pk_ref), 0, right)
        h1l = start_gather(1, (x_ref, pk_ref), 1, left)
        compute_stage(-1)
        for r in h1r + h1l:
            r.wait()
        h2 = start_gather(2, (x_all.at[0], pk_all.at[0]), 2, right)
        compute_stage(0)
        pa = start_partial(0, 0, left)
        compute_stage(1)
        pb = start_partial(1, 2, right)
        for r in h2:
            r.wait()
        compute_stage(2)
        pc = start_partial(2, 1, diag)
        for r in pa + pb + pc:
            r.wait()

        def sum_step(t, _):
            rows = pl.ds(t * TT, TT)
            acc = out_ref[rows, :].astype(jnp.float32)
            for k in range(N_DEV - 1):
                acc = acc + (precv[k, rows, :].astype(jnp.float32)
                             * pscale_recv[k, rows, :].astype(jnp.float32))
            out_ref[rows, :] = acc.astype(jnp.bfloat16)
            return _

        lax.fori_loop(0, N_TILES, sum_step, 0)

    out_bf = pl.pallas_call(
        body,
        out_shape=jax.ShapeDtypeStruct((T, D), jnp.bfloat16),
        in_specs=[
            pl.BlockSpec(memory_space=pltpu.VMEM),
            pl.BlockSpec(memory_space=pltpu.VMEM),
            pl.BlockSpec(memory_space=pltpu.MemorySpace.HBM),
        ],
        out_specs=pl.BlockSpec(memory_space=pltpu.VMEM),
        scratch_shapes=[
            pltpu.VMEM((N_DEV - 1, T, D), jnp.bfloat16),
            pltpu.VMEM((N_DEV - 1, T, 4), jnp.bfloat16),
            pltpu.VMEM((E_LOCAL * D, D), jnp.bfloat16),
            pltpu.VMEM((2, TT, E_LOCAL * D), jnp.bfloat16),
            pltpu.VMEM((N_DEV - 1, T, D), jnp.int8),
            pltpu.VMEM((N_DEV - 1, T, 1), jnp.bfloat16),
            pltpu.VMEM((N_DEV - 1, T, D), jnp.int8),
            pltpu.VMEM((N_DEV - 1, T, 1), jnp.bfloat16),
            pltpu.SemaphoreType.DMA((N_DEV - 1, 2)),
            pltpu.SemaphoreType.DMA((N_DEV - 1, 2)),
            pltpu.SemaphoreType.DMA((N_DEV - 1, 2)),
            pltpu.SemaphoreType.DMA((N_DEV - 1, 2)),
            pltpu.SemaphoreType.DMA((2,)),
        ],
        compiler_params=pltpu.CompilerParams(
            collective_id=0,
            vmem_limit_bytes=128 * 1024 * 1024,
        ),
    )(x_bf, packed, ew_bf)
    return out_bf
